# Initial kernel scaffold; baseline (speedup 1.0000x reference)
import jax
import jax.numpy as jnp
from jax.experimental import pallas as pl


def kernel(x, edge_index, edge_attr, We, be, W1, b1, W2, b2):
    raise NotImplementedError("write your pallas kernel here")


# speedup vs baseline: 3.2468x; 3.2468x over previous
import functools

import jax
import jax.numpy as jnp
from jax import lax
from jax.experimental import pallas as pl
from jax.experimental.pallas import tpu as pltpu
from jax.experimental.pallas import tpu_sc as plsc

N = 10000
E = 320000
D = 128
D_EDGE = 16
DEPTH = 3

NC = 2
NS = 16
NW = NC * NS
EPW = E // NW
CH = 80
NCHUNK = EPW // CH
SCH = 25
NSCH = NCHUNK // SCH
WB = 80
NWBC = N // WB
WB_PER_TILE = -(-NWBC // NS)


def _sc_agg_body(h_hbm, e_hbm, src_hbm, dst_hbm, out_hbm,
                 src_v, dst_v, hbuf, ebuf, acc, sem):
    cid = lax.axis_index("c")
    sid = lax.axis_index("s")

    zv = jnp.zeros((16,), jnp.float32)

    def zrow(i, carry):
        for j in range(D // 16):
            hbuf[i, pl.ds(j * 16, 16)] = zv
        return carry

    lax.fori_loop(0, WB, zrow, 0)
    for k in range(WB_PER_TILE):
        c = sid + NS * k

        @pl.when(c < NWBC)
        def _():
            pltpu.sync_copy(hbuf, acc.at[pl.ds(c * WB, WB)])

    plsc.subcore_barrier()

    wid = cid * NS + sid

    def superchunk(u, carry):
        pltpu.sync_copy(src_hbm.at[wid, u], src_v)
        pltpu.sync_copy(dst_hbm.at[wid, u], dst_v)

        def chunk(t, c1):
            off = wid * EPW + (u * SCH + t) * CH
            pltpu.async_copy(h_hbm.at[src_v.at[t]], hbuf, sem).wait()
            pltpu.sync_copy(e_hbm.at[pl.ds(off, CH)], ebuf)

            def crow(i, c2):
                for j in range(D // 16):
                    s = pl.ds(j * 16, 16)
                    hbuf[i, s] = jnp.maximum(hbuf[i, s] + ebuf[i, s], 0.0)
                return c2

            lax.fori_loop(0, CH, crow, 0)
            pltpu.sync_copy(hbuf, acc.at[dst_v.at[t]], add=True)
            return c1

        lax.fori_loop(0, SCH, chunk, 0)
        return carry

    lax.fori_loop(0, NSCH, superchunk, 0)
    plsc.subcore_barrier()

    for k in range(WB_PER_TILE):
        c = sid + NS * k

        @pl.when(c < NWBC)
        def _():
            pltpu.sync_copy(acc.at[pl.ds(c * WB, WB)], hbuf)
            pltpu.sync_copy(hbuf, out_hbm.at[cid, pl.ds(c * WB, WB)])


_sc_agg = functools.partial(
    pl.kernel,
    out_type=jax.ShapeDtypeStruct((NC, N, D), jnp.float32),
    mesh=plsc.VectorSubcoreMesh(
        core_axis_name="c", subcore_axis_name="s",
        num_cores=NC, num_subcores=NS),
    scratch_types=[
        pltpu.VMEM((SCH, CH), jnp.int32),
        pltpu.VMEM((SCH, CH), jnp.int32),
        pltpu.VMEM((CH, D), jnp.float32),
        pltpu.VMEM((CH, D), jnp.float32),
        pltpu.VMEM_SHARED((N, D), jnp.float32),
        pltpu.SemaphoreType.DMA,
    ],
)(_sc_agg_body)


def _eproj_body(ea_ref, we_ref, be_ref, out_ref):
    out_ref[...] = (
        jnp.dot(ea_ref[...], we_ref[...], preferred_element_type=jnp.float32)
        + be_ref[...]
    )


def _eproj(edge_attr, We, be2d):
    blk = 3200
    return pl.pallas_call(
        _eproj_body,
        grid=(E // blk,),
        in_specs=[
            pl.BlockSpec((blk, D_EDGE), lambda i: (i, 0)),
            pl.BlockSpec((D_EDGE, D), lambda i: (0, 0)),
            pl.BlockSpec((1, D), lambda i: (0, 0)),
        ],
        out_specs=pl.BlockSpec((blk, D), lambda i: (i, 0)),
        out_shape=jax.ShapeDtypeStruct((E, D), jnp.float32),
    )(edge_attr, We, be2d)


def _mlp_body(h_ref, agg_ref, w1_ref, b1_ref, w2_ref, b2_ref, out_ref):
    z = h_ref[...] + agg_ref[0] + agg_ref[1]
    z = jnp.maximum(
        jnp.dot(z, w1_ref[...], preferred_element_type=jnp.float32)
        + b1_ref[...], 0.0)
    out_ref[...] = (
        jnp.dot(z, w2_ref[...], preferred_element_type=jnp.float32)
        + b2_ref[...]
    )


def _mlp(h, agg, W1l, b1l, W2l, b2l):
    blk = 2000
    return pl.pallas_call(
        _mlp_body,
        grid=(N // blk,),
        in_specs=[
            pl.BlockSpec((blk, D), lambda i: (i, 0)),
            pl.BlockSpec((NC, blk, D), lambda i: (0, i, 0)),
            pl.BlockSpec((D, D), lambda i: (0, 0)),
            pl.BlockSpec((1, D), lambda i: (0, 0)),
            pl.BlockSpec((D, D), lambda i: (0, 0)),
            pl.BlockSpec((1, D), lambda i: (0, 0)),
        ],
        out_specs=pl.BlockSpec((blk, D), lambda i: (i, 0)),
        out_shape=jax.ShapeDtypeStruct((N, D), jnp.float32),
    )(h, agg, W1l, b1l, W2l, b2l)


def kernel(x, edge_index, edge_attr, We, be, W1, b1, W2, b2):
    src2d = edge_index[0].reshape(NW, NSCH, SCH, CH)
    dst2d = edge_index[1].reshape(NW, NSCH, SCH, CH)
    e = _eproj(edge_attr, We, be.reshape(1, D))
    h = x
    for l in range(DEPTH):
        agg = _sc_agg(h, e, src2d, dst2d)
        h = _mlp(h, agg, W1[l], b1[l].reshape(1, D),
                 W2[l], b2[l].reshape(1, D))
    return h


# speedup vs baseline: 4.4878x; 1.3822x over previous
import functools

import jax
import jax.numpy as jnp
from jax import lax
from jax.experimental import pallas as pl
from jax.experimental.pallas import tpu as pltpu
from jax.experimental.pallas import tpu_sc as plsc

N = 10000
E = 320000
D = 128
D_EDGE = 16
DEPTH = 3

NC = 2
NS = 16
NW = NC * NS
EPW = E // NW
CH = 80
NCHUNK = EPW // CH
SCH = 25
NSCH = NCHUNK // SCH
WB = 80
NWBC = N // WB
WB_PER_TILE = -(-NWBC // NS)


def _sc_agg_body(h_hbm, e_hbm, src_hbm, dst_hbm, out_hbm,
                 src_v, dst_v, hA, hB, eA, eB, acc,
                 semEA, semEB, semGA, semGB, semSA, semSB):
    cid = lax.axis_index("c")
    sid = lax.axis_index("s")

    zv = jnp.zeros((16,), jnp.float32)

    def zrow(i, carry):
        for j in range(D // 16):
            hA[i, pl.ds(j * 16, 16)] = zv
        return carry

    lax.fori_loop(0, WB, zrow, 0)
    for k in range(WB_PER_TILE):
        c = sid + NS * k

        @pl.when(c < NWBC)
        def _():
            pltpu.sync_copy(hA, acc.at[pl.ds(c * WB, WB)])

    plsc.subcore_barrier()

    wid = cid * NS + sid

    def relu_rows(hbuf, ebuf):
        def crow(i, c2):
            for j in range(D // 16):
                s = pl.ds(j * 16, 16)
                hbuf[i, s] = jnp.maximum(hbuf[i, s] + ebuf[i, s], 0.0)
            return c2

        lax.fori_loop(0, CH, crow, 0)

    def superchunk(u, carry):
        pltpu.sync_copy(src_hbm.at[wid, u], src_v)
        pltpu.sync_copy(dst_hbm.at[wid, u], dst_v)
        gbase = wid * NCHUNK + u * SCH

        def issue_e(t, ebuf, sem):
            pltpu.async_copy(
                e_hbm.at[pl.ds((gbase + t) * CH, CH)], ebuf, sem)

        def wait_e(ebuf, sem):
            pltpu.make_async_copy(
                e_hbm.at[pl.ds(gbase * CH, CH)], ebuf, sem).wait()

        def issue_g(t, hbuf, sem):
            pltpu.async_copy(h_hbm.at[src_v.at[t]], hbuf, sem)

        def wait_g(hbuf, sem):
            pltpu.make_async_copy(h_hbm.at[src_v.at[0]], hbuf, sem).wait()

        def issue_s(t, hbuf, sem):
            pltpu.async_copy(hbuf, acc.at[dst_v.at[t]], sem, add=True)

        def wait_s(hbuf, sem):
            pltpu.make_async_copy(hbuf, acc.at[dst_v.at[0]], sem).wait()

        issue_e(0, eA, semEA)
        issue_g(0, hA, semGA)
        issue_e(1, eB, semEB)
        issue_g(1, hB, semGB)

        def pair(p, c1):
            tA = 2 * p
            wait_e(eA, semEA)
            wait_g(hA, semGA)
            relu_rows(hA, eA)
            issue_s(tA, hA, semSA)
            issue_e(tA + 2, eA, semEA)

            @pl.when(p > 0)
            def _():
                wait_s(hB, semSB)
                issue_g(tA + 1, hB, semGB)

            tB = tA + 1
            wait_e(eB, semEB)
            wait_g(hB, semGB)
            relu_rows(hB, eB)
            issue_s(tB, hB, semSB)

            @pl.when(p < SCH // 2 - 1)
            def _():
                issue_e(tB + 2, eB, semEB)

            wait_s(hA, semSA)
            issue_g(tB + 1, hA, semGA)
            return c1

        lax.fori_loop(0, SCH // 2, pair, 0)

        wait_e(eA, semEA)
        wait_g(hA, semGA)
        relu_rows(hA, eA)
        issue_s(SCH - 1, hA, semSA)
        wait_s(hB, semSB)
        wait_s(hA, semSA)
        return carry

    lax.fori_loop(0, NSCH, superchunk, 0)
    plsc.subcore_barrier()

    for k in range(WB_PER_TILE):
        c = sid + NS * k

        @pl.when(c < NWBC)
        def _():
            pltpu.sync_copy(acc.at[pl.ds(c * WB, WB)], hA)
            pltpu.sync_copy(hA, out_hbm.at[cid, pl.ds(c * WB, WB)])


_sc_agg = functools.partial(
    pl.kernel,
    out_type=jax.ShapeDtypeStruct((NC, N, D), jnp.float32),
    mesh=plsc.VectorSubcoreMesh(
        core_axis_name="c", subcore_axis_name="s",
        num_cores=NC, num_subcores=NS),
    scratch_types=[
        pltpu.VMEM((SCH, CH), jnp.int32),
        pltpu.VMEM((SCH, CH), jnp.int32),
        pltpu.VMEM((CH, D), jnp.float32),
        pltpu.VMEM((CH, D), jnp.float32),
        pltpu.VMEM((CH, D), jnp.float32),
        pltpu.VMEM((CH, D), jnp.float32),
        pltpu.VMEM_SHARED((N, D), jnp.float32),
        pltpu.SemaphoreType.DMA,
        pltpu.SemaphoreType.DMA,
        pltpu.SemaphoreType.DMA,
        pltpu.SemaphoreType.DMA,
        pltpu.SemaphoreType.DMA,
        pltpu.SemaphoreType.DMA,
    ],
)(_sc_agg_body)


def _eproj_body(ea_ref, we_ref, be_ref, out_ref):
    out_ref[...] = (
        jnp.dot(ea_ref[...], we_ref[...], preferred_element_type=jnp.float32)
        + be_ref[...]
    )


def _eproj(edge_attr, We, be2d):
    blk = 3200
    return pl.pallas_call(
        _eproj_body,
        grid=(E // blk,),
        in_specs=[
            pl.BlockSpec((blk, D_EDGE), lambda i: (i, 0)),
            pl.BlockSpec((D_EDGE, D), lambda i: (0, 0)),
            pl.BlockSpec((1, D), lambda i: (0, 0)),
        ],
        out_specs=pl.BlockSpec((blk, D), lambda i: (i, 0)),
        out_shape=jax.ShapeDtypeStruct((E, D), jnp.float32),
    )(edge_attr, We, be2d)


def _mlp_body(h_ref, agg_ref, w1_ref, b1_ref, w2_ref, b2_ref, out_ref):
    z = h_ref[...] + agg_ref[0] + agg_ref[1]
    z = jnp.maximum(
        jnp.dot(z, w1_ref[...], preferred_element_type=jnp.float32)
        + b1_ref[...], 0.0)
    out_ref[...] = (
        jnp.dot(z, w2_ref[...], preferred_element_type=jnp.float32)
        + b2_ref[...]
    )


def _mlp(h, agg, W1l, b1l, W2l, b2l):
    blk = 2000
    return pl.pallas_call(
        _mlp_body,
        grid=(N // blk,),
        in_specs=[
            pl.BlockSpec((blk, D), lambda i: (i, 0)),
            pl.BlockSpec((NC, blk, D), lambda i: (0, i, 0)),
            pl.BlockSpec((D, D), lambda i: (0, 0)),
            pl.BlockSpec((1, D), lambda i: (0, 0)),
            pl.BlockSpec((D, D), lambda i: (0, 0)),
            pl.BlockSpec((1, D), lambda i: (0, 0)),
        ],
        out_specs=pl.BlockSpec((blk, D), lambda i: (i, 0)),
        out_shape=jax.ShapeDtypeStruct((N, D), jnp.float32),
    )(h, agg, W1l, b1l, W2l, b2l)


def kernel(x, edge_index, edge_attr, We, be, W1, b1, W2, b2):
    src2d = edge_index[0].reshape(NW, NSCH, SCH, CH)
    dst2d = edge_index[1].reshape(NW, NSCH, SCH, CH)
    e = _eproj(edge_attr, We, be.reshape(1, D))
    h = x
    for l in range(DEPTH):
        agg = _sc_agg(h, e, src2d, dst2d)
        h = _mlp(h, agg, W1[l], b1[l].reshape(1, D),
                 W2[l], b2[l].reshape(1, D))
    return h


# speedup vs baseline: 5.9353x; 1.3225x over previous
import functools

import jax
import jax.numpy as jnp
from jax import lax
from jax.experimental import pallas as pl
from jax.experimental.pallas import tpu as pltpu
from jax.experimental.pallas import tpu_sc as plsc

N = 10000
E = 320000
D = 128
D_EDGE = 16
DEPTH = 3

NC = 2
NS = 16
NW = NC * NS
EPW = E // NW
CH = 80
NCHUNK = EPW // CH
SCH = 25
NSCH = NCHUNK // SCH
WB = 80
NWBC = N // WB
WB_PER_TILE = -(-NWBC // NS)


def _sc_agg_body(h_hbm, e_hbm, src_hbm, dst_hbm, out_hbm,
                 idx_v, hbuf3, ebuf2, acc, *sems):
    cid = lax.axis_index("c")
    sid = lax.axis_index("s")
    src_v = idx_v.at[0]
    dst_v = idx_v.at[1]
    hA = hbuf3.at[0]
    hbufs = (hbuf3.at[0], hbuf3.at[1], hbuf3.at[2])
    ebufs = (ebuf2.at[0], ebuf2.at[1])
    semE, semG, semS = sems[0:2], sems[2:5], sems[5:8]

    zv = jnp.zeros((16,), jnp.float32)

    def zrow(i, carry):
        for j in range(D // 16):
            hA[i, pl.ds(j * 16, 16)] = zv
        return carry

    lax.fori_loop(0, WB, zrow, 0)
    for k in range(WB_PER_TILE):
        c = sid + NS * k

        @pl.when(c < NWBC)
        def _():
            pltpu.sync_copy(hA, acc.at[pl.ds(c * WB, WB)])

    plsc.subcore_barrier()

    wid = cid * NS + sid

    def relu_rows(hbuf, ebuf):
        @plsc.parallel_loop(0, CH // 2, unroll=2)
        def crow(i2):
            ia = 2 * i2
            ib = 2 * i2 + 1
            for j in range(D // 16):
                s = pl.ds(j * 16, 16)
                w = lax.bitcast_convert_type(ebuf[i2, s], jnp.int32)
                a = lax.bitcast_convert_type(w << 16, jnp.float32)
                b = lax.bitcast_convert_type(w & jnp.int32(-65536),
                                             jnp.float32)
                hbuf[ia, s] = jnp.maximum(hbuf[ia, s] + a, 0.0)
                hbuf[ib, s] = jnp.maximum(hbuf[ib, s] + b, 0.0)

    def superchunk(u, carry):
        pltpu.sync_copy(src_hbm.at[wid, u], src_v)
        pltpu.sync_copy(dst_hbm.at[wid, u], dst_v)
        gbase = wid * NCHUNK + u * SCH

        def issue_e(t, q):
            pltpu.async_copy(
                e_hbm.at[pl.ds((gbase + t) * (CH // 2), CH // 2)],
                ebufs[q], semE[q])

        def wait_e(q):
            pltpu.make_async_copy(
                e_hbm.at[pl.ds(gbase * (CH // 2), CH // 2)],
                ebufs[q], semE[q]).wait()

        def issue_g(t, q):
            pltpu.async_copy(h_hbm.at[src_v.at[t]], hbufs[q], semG[q])

        def wait_g(q):
            pltpu.make_async_copy(
                h_hbm.at[src_v.at[0]], hbufs[q], semG[q]).wait()

        def issue_s(t, q):
            pltpu.async_copy(hbufs[q], acc.at[dst_v.at[t]], semS[q],
                             add=True)

        def wait_s(q):
            pltpu.make_async_copy(hbufs[q], acc.at[dst_v.at[0]],
                                  semS[q]).wait()

        issue_e(0, 0)
        issue_e(1, 1)
        issue_g(0, 0)
        issue_g(1, 1)

        def six(p, c1):
            for q in range(6):
                c = 6 * p + q
                hq = q % 3
                eq = q % 2
                wait_e(eq)
                wait_g(hq)
                relu_rows(hbufs[hq], ebufs[eq])
                issue_s(c, hq)
                z = (q + 2) % 3

                if q == 0:
                    @pl.when(p > 0)
                    def _():
                        wait_s(z)
                else:
                    wait_s(z)

                if q == 5:
                    @pl.when(p < SCH // 6 - 1)
                    def _():
                        issue_g(c + 2, z)
                        issue_e(c + 2, eq)
                else:
                    issue_g(c + 2, z)
                    issue_e(c + 2, eq)
            return c1

        lax.fori_loop(0, SCH // 6, six, 0)

        wait_e(0)
        wait_g(0)
        relu_rows(hbufs[0], ebufs[0])
        issue_s(SCH - 1, 0)
        wait_s(2)
        wait_s(0)
        return carry

    lax.fori_loop(0, NSCH, superchunk, 0)
    plsc.subcore_barrier()

    for k in range(WB_PER_TILE):
        c = sid + NS * k

        @pl.when(c < NWBC)
        def _():
            pltpu.sync_copy(acc.at[pl.ds(c * WB, WB)], hA)
            pltpu.sync_copy(hA, out_hbm.at[cid, pl.ds(c * WB, WB)])


_sc_agg = functools.partial(
    pl.kernel,
    out_type=jax.ShapeDtypeStruct((NC, N, D), jnp.float32),
    mesh=plsc.VectorSubcoreMesh(
        core_axis_name="c", subcore_axis_name="s",
        num_cores=NC, num_subcores=NS),
    scratch_types=[
        pltpu.VMEM((2, SCH, CH), jnp.int32),
        pltpu.VMEM((3, CH, D), jnp.float32),
        pltpu.VMEM((2, CH // 2, D), jnp.float32),
        pltpu.VMEM_SHARED((N, D), jnp.float32),
    ] + [pltpu.SemaphoreType.DMA] * 8,
)(_sc_agg_body)


def _eproj_body(ea_ref, we_ref, be_ref, out_ref):
    blk = ea_ref.shape[0]
    z = (jnp.dot(ea_ref[...], we_ref[...],
                 preferred_element_type=jnp.float32) + be_ref[...])
    u = lax.bitcast_convert_type(z, jnp.int32)
    t = lax.shift_right_logical(u + jnp.int32(0x8000), 16)
    tr = t.reshape(blk // 2, 2, D)
    w = tr[:, 0, :] | (tr[:, 1, :] << 16)
    out_ref[...] = lax.bitcast_convert_type(w, jnp.float32)


def _eproj(edge_attr, We, be2d):
    blk = 3200
    return pl.pallas_call(
        _eproj_body,
        grid=(E // blk,),
        in_specs=[
            pl.BlockSpec((blk, D_EDGE), lambda i: (i, 0)),
            pl.BlockSpec((D_EDGE, D), lambda i: (0, 0)),
            pl.BlockSpec((1, D), lambda i: (0, 0)),
        ],
        out_specs=pl.BlockSpec((blk // 2, D), lambda i: (i, 0)),
        out_shape=jax.ShapeDtypeStruct((E // 2, D), jnp.float32),
    )(edge_attr, We, be2d)


def _mlp_body(h_ref, agg_ref, w1_ref, b1_ref, w2_ref, b2_ref, out_ref):
    z = h_ref[...] + agg_ref[0] + agg_ref[1]
    z = jnp.maximum(
        jnp.dot(z, w1_ref[...], preferred_element_type=jnp.float32)
        + b1_ref[...], 0.0)
    out_ref[...] = (
        jnp.dot(z, w2_ref[...], preferred_element_type=jnp.float32)
        + b2_ref[...]
    )


def _mlp(h, agg, W1l, b1l, W2l, b2l):
    blk = 2000
    return pl.pallas_call(
        _mlp_body,
        grid=(N // blk,),
        in_specs=[
            pl.BlockSpec((blk, D), lambda i: (i, 0)),
            pl.BlockSpec((NC, blk, D), lambda i: (0, i, 0)),
            pl.BlockSpec((D, D), lambda i: (0, 0)),
            pl.BlockSpec((1, D), lambda i: (0, 0)),
            pl.BlockSpec((D, D), lambda i: (0, 0)),
            pl.BlockSpec((1, D), lambda i: (0, 0)),
        ],
        out_specs=pl.BlockSpec((blk, D), lambda i: (i, 0)),
        out_shape=jax.ShapeDtypeStruct((N, D), jnp.float32),
    )(h, agg, W1l, b1l, W2l, b2l)


def kernel(x, edge_index, edge_attr, We, be, W1, b1, W2, b2):
    src2d = edge_index[0].reshape(NW, NSCH, SCH, CH)
    dst2d = edge_index[1].reshape(NW, NSCH, SCH, CH)
    e = _eproj(edge_attr, We, be.reshape(1, D))
    h = x
    for l in range(DEPTH):
        agg = _sc_agg(h, e, src2d, dst2d)
        h = _mlp(h, agg, W1[l], b1[l].reshape(1, D),
                 W2[l], b2[l].reshape(1, D))
    return h


# speedup vs baseline: 5.9985x; 1.0106x over previous
import functools

import jax
import jax.numpy as jnp
from jax import lax
from jax.experimental import pallas as pl
from jax.experimental.pallas import tpu as pltpu
from jax.experimental.pallas import tpu_sc as plsc

N = 10000
E = 320000
D = 128
D_EDGE = 16
DEPTH = 3

NC = 2
NS = 16
NW = NC * NS
EPW = E // NW
CH = 80
NCHUNK = EPW // CH
SCH = 25
NSCH = NCHUNK // SCH
WB = 80
NWBC = N // WB
WB_PER_TILE = -(-NWBC // NS)


def _sc_agg_body(h_hbm, e_hbm, src_hbm, dst_hbm, out_hbm,
                 idx_v, hbuf3, ebuf2, acc, *sems):
    cid = lax.axis_index("c")
    sid = lax.axis_index("s")
    src_v = idx_v.at[0]
    dst_v = idx_v.at[1]
    hA = hbuf3.at[0]
    hbufs = (hbuf3.at[0], hbuf3.at[1], hbuf3.at[2])
    ebufs = (ebuf2.at[0], ebuf2.at[1])
    semE, semG, semS = sems[0:2], sems[2:5], sems[5:8]

    zv = jnp.zeros((16,), jnp.float32)

    def zrow(i, carry):
        for j in range(D // 16):
            hA[i, pl.ds(j * 16, 16)] = zv
        return carry

    lax.fori_loop(0, WB, zrow, 0)
    for k in range(WB_PER_TILE):
        c = sid + NS * k

        @pl.when(c < NWBC)
        def _():
            pltpu.sync_copy(hA, acc.at[pl.ds(c * WB, WB)])

    plsc.subcore_barrier()

    wid = cid * NS + sid

    def relu_rows(hbuf, ebuf):
        @plsc.parallel_loop(0, CH // 2, unroll=4)
        def crow(i2):
            ia = 2 * i2
            ib = 2 * i2 + 1
            for j in range(D // 16):
                s = pl.ds(j * 16, 16)
                w = lax.bitcast_convert_type(ebuf[i2, s], jnp.int32)
                a = lax.bitcast_convert_type(w << 16, jnp.float32)
                b = lax.bitcast_convert_type(w & jnp.int32(-65536),
                                             jnp.float32)
                hbuf[ia, s] = jnp.maximum(hbuf[ia, s] + a, 0.0)
                hbuf[ib, s] = jnp.maximum(hbuf[ib, s] + b, 0.0)

    def superchunk(u, carry):
        pltpu.sync_copy(src_hbm.at[wid, u], src_v)
        pltpu.sync_copy(dst_hbm.at[wid, u], dst_v)
        gbase = wid * NCHUNK + u * SCH

        def issue_e(t, q):
            pltpu.async_copy(
                e_hbm.at[pl.ds((gbase + t) * (CH // 2), CH // 2)],
                ebufs[q], semE[q])

        def wait_e(q):
            pltpu.make_async_copy(
                e_hbm.at[pl.ds(gbase * (CH // 2), CH // 2)],
                ebufs[q], semE[q]).wait()

        def issue_g(t, q):
            pltpu.async_copy(h_hbm.at[src_v.at[t]], hbufs[q], semG[q])

        def wait_g(q):
            pltpu.make_async_copy(
                h_hbm.at[src_v.at[0]], hbufs[q], semG[q]).wait()

        def issue_s(t, q):
            pltpu.async_copy(hbufs[q], acc.at[dst_v.at[t]], semS[q],
                             add=True)

        def wait_s(q):
            pltpu.make_async_copy(hbufs[q], acc.at[dst_v.at[0]],
                                  semS[q]).wait()

        issue_e(0, 0)
        issue_e(1, 1)
        issue_g(0, 0)
        issue_g(1, 1)

        def six(p, c1):
            for q in range(6):
                c = 6 * p + q
                hq = q % 3
                eq = q % 2
                wait_e(eq)
                wait_g(hq)
                relu_rows(hbufs[hq], ebufs[eq])
                issue_s(c, hq)
                z = (q + 2) % 3

                if q == 0:
                    @pl.when(p > 0)
                    def _():
                        wait_s(z)
                else:
                    wait_s(z)

                if q == 5:
                    @pl.when(p < SCH // 6 - 1)
                    def _():
                        issue_g(c + 2, z)
                        issue_e(c + 2, eq)
                else:
                    issue_g(c + 2, z)
                    issue_e(c + 2, eq)
            return c1

        lax.fori_loop(0, SCH // 6, six, 0)

        wait_e(0)
        wait_g(0)
        relu_rows(hbufs[0], ebufs[0])
        issue_s(SCH - 1, 0)
        wait_s(2)
        wait_s(0)
        return carry

    lax.fori_loop(0, NSCH, superchunk, 0)
    plsc.subcore_barrier()

    for k in range(WB_PER_TILE):
        c = sid + NS * k

        @pl.when(c < NWBC)
        def _():
            pltpu.sync_copy(acc.at[pl.ds(c * WB, WB)], hA)
            pltpu.sync_copy(hA, out_hbm.at[cid, pl.ds(c * WB, WB)])


_sc_agg = functools.partial(
    pl.kernel,
    out_type=jax.ShapeDtypeStruct((NC, N, D), jnp.float32),
    mesh=plsc.VectorSubcoreMesh(
        core_axis_name="c", subcore_axis_name="s",
        num_cores=NC, num_subcores=NS),
    scratch_types=[
        pltpu.VMEM((2, SCH, CH), jnp.int32),
        pltpu.VMEM((3, CH, D), jnp.float32),
        pltpu.VMEM((2, CH // 2, D), jnp.float32),
        pltpu.VMEM_SHARED((N, D), jnp.float32),
    ] + [pltpu.SemaphoreType.DMA] * 8,
)(_sc_agg_body)


def _eproj_body(ea_ref, we_ref, be_ref, out_ref):
    blk = ea_ref.shape[0]
    z = (jnp.dot(ea_ref[...], we_ref[...],
                 preferred_element_type=jnp.float32) + be_ref[...])
    u = lax.bitcast_convert_type(z, jnp.int32)
    t = lax.shift_right_logical(u + jnp.int32(0x8000), 16)
    tr = t.reshape(blk // 2, 2, D)
    w = tr[:, 0, :] | (tr[:, 1, :] << 16)
    out_ref[...] = lax.bitcast_convert_type(w, jnp.float32)


def _eproj(edge_attr, We, be2d):
    blk = 3200
    return pl.pallas_call(
        _eproj_body,
        grid=(E // blk,),
        in_specs=[
            pl.BlockSpec((blk, D_EDGE), lambda i: (i, 0)),
            pl.BlockSpec((D_EDGE, D), lambda i: (0, 0)),
            pl.BlockSpec((1, D), lambda i: (0, 0)),
        ],
        out_specs=pl.BlockSpec((blk // 2, D), lambda i: (i, 0)),
        out_shape=jax.ShapeDtypeStruct((E // 2, D), jnp.float32),
    )(edge_attr, We, be2d)


def _mlp_body(h_ref, agg_ref, w1_ref, b1_ref, w2_ref, b2_ref, out_ref):
    z = h_ref[...] + agg_ref[0] + agg_ref[1]
    z = jnp.maximum(
        jnp.dot(z, w1_ref[...], preferred_element_type=jnp.float32)
        + b1_ref[...], 0.0)
    out_ref[...] = (
        jnp.dot(z, w2_ref[...], preferred_element_type=jnp.float32)
        + b2_ref[...]
    )


def _mlp(h, agg, W1l, b1l, W2l, b2l):
    blk = 2000
    return pl.pallas_call(
        _mlp_body,
        grid=(N // blk,),
        in_specs=[
            pl.BlockSpec((blk, D), lambda i: (i, 0)),
            pl.BlockSpec((NC, blk, D), lambda i: (0, i, 0)),
            pl.BlockSpec((D, D), lambda i: (0, 0)),
            pl.BlockSpec((1, D), lambda i: (0, 0)),
            pl.BlockSpec((D, D), lambda i: (0, 0)),
            pl.BlockSpec((1, D), lambda i: (0, 0)),
        ],
        out_specs=pl.BlockSpec((blk, D), lambda i: (i, 0)),
        out_shape=jax.ShapeDtypeStruct((N, D), jnp.float32),
    )(h, agg, W1l, b1l, W2l, b2l)


def kernel(x, edge_index, edge_attr, We, be, W1, b1, W2, b2):
    src2d = edge_index[0].reshape(NW, NSCH, SCH, CH)
    dst2d = edge_index[1].reshape(NW, NSCH, SCH, CH)
    e = _eproj(edge_attr, We, be.reshape(1, D))
    h = x
    for l in range(DEPTH):
        agg = _sc_agg(h, e, src2d, dst2d)
        h = _mlp(h, agg, W1[l], b1[l].reshape(1, D),
                 W2[l], b2[l].reshape(1, D))
    return h
